# baseline (speedup 1.0000x reference)
import functools

import jax
import jax.numpy as jnp
from jax import lax
from jax.experimental import pallas as pl
from jax.experimental.pallas import tpu as pltpu
from jax.experimental.pallas import tpu_sc as plsc

N_NODES = 10020
D = 128
N_PAD = 10240
N_TILES = 32
RPT = N_PAD // 16
E = 320640
CHUNK = 128
NBUF = 4
_NCH_MIN = -(-E // (N_TILES * CHUNK))
NCH = -(-_NCH_MIN // NBUF) * NBUF
E_PAD = N_TILES * CHUNK * NCH
EPT = E_PAD // N_TILES
SCHUNK = 64
NPASS = 4
CH_A = 60
CH_B = 20
_TOT_CH = 16 * NPASS * (CH_A + CH_B) * SCHUNK
assert _TOT_CH == E_PAD, (_TOT_CH, E_PAD)
NUM_GRAPHS = 334
NODES_PER_GRAPH = 30
IDX_PAD = 16384
UV = 2 * IDX_PAD
UV_PT = UV // N_TILES
UV_NCH = UV_PT // CHUNK

_MESH = plsc.VectorSubcoreMesh(core_axis_name="c", subcore_axis_name="s")


@functools.partial(
    pl.kernel,
    out_type=jax.ShapeDtypeStruct((2 * N_PAD, D), jnp.float32),
    mesh=_MESH,
    scratch_types=[
        pltpu.VMEM((NCH, CHUNK), jnp.int32),
        pltpu.VMEM((CHUNK, D), jnp.float32),
        pltpu.VMEM_SHARED((N_PAD, D), jnp.float32),
        pltpu.SemaphoreType.DMA,
    ],
)
def _sc_deg(dstp3, onesr, zrows, out, dst_all, ones_v, acc_sh, ssem):
    cid = lax.axis_index("c")
    sid = lax.axis_index("s")
    wid = sid * 2 + cid
    pltpu.sync_copy(zrows, acc_sh.at[pl.ds(sid * RPT, RPT)])
    pltpu.sync_copy(dstp3.at[wid], dst_all)
    pltpu.sync_copy(onesr, ones_v)
    plsc.subcore_barrier()

    def fire(t, carry):
        pltpu.async_copy(ones_v, acc_sh.at[dst_all.at[t]], ssem, add=True)
        return carry

    lax.fori_loop(0, NCH, fire, 0)

    def drain(t, carry):
        pltpu.make_async_copy(ones_v, acc_sh.at[dst_all.at[0]], ssem).wait()
        return carry

    lax.fori_loop(0, NCH, drain, 0)
    plsc.subcore_barrier()
    pltpu.sync_copy(acc_sh.at[pl.ds(sid * RPT, RPT)],
                    out.at[pl.ds(cid * N_PAD + sid * RPT, RPT)])


@functools.partial(
    pl.kernel,
    out_type=jax.ShapeDtypeStruct((2 * N_PAD, D), jnp.float32),
    mesh=_MESH,
    scratch_types=[
        pltpu.VMEM((CH_A, SCHUNK), jnp.int32),
        pltpu.VMEM((CH_A, SCHUNK), jnp.int32),
        pltpu.VMEM((SCHUNK, D), jnp.float32),
        pltpu.VMEM((SCHUNK, D), jnp.float32),
        pltpu.VMEM((SCHUNK, D), jnp.float32),
        pltpu.VMEM((SCHUNK, D), jnp.float32),
        pltpu.VMEM_SHARED((N_PAD, D), jnp.float32),
        pltpu.SemaphoreType.DMA,
        pltpu.SemaphoreType.DMA,
        pltpu.SemaphoreType.DMA,
        pltpu.SemaphoreType.DMA,
        pltpu.SemaphoreType.DMA,
        pltpu.SemaphoreType.DMA,
        pltpu.SemaphoreType.DMA,
        pltpu.SemaphoreType.DMA,
    ],
)
def _sc_scatter(hs, srcp4, dstp4, zrows, out, src_all, dst_all,
                r0, r1, r2, r3, acc_sh, g0, g1, g2, g3, s0, s1, s2, s3):
    rows = (r0, r1, r2, r3)
    gsem = (g0, g1, g2, g3)
    ssem = (s0, s1, s2, s3)
    cid = lax.axis_index("c")
    sid = lax.axis_index("s")
    wid = sid * 2 + cid
    nch = lax.select(cid == 0, CH_A, CH_B)
    ntrip = lax.select(cid == 0, CH_A // NBUF, CH_B // NBUF)
    pltpu.sync_copy(zrows, acc_sh.at[pl.ds(sid * RPT, RPT)])
    plsc.subcore_barrier()

    for p in range(NPASS):
        pltpu.sync_copy(srcp4.at[wid, p], src_all)
        pltpu.sync_copy(dstp4.at[wid, p], dst_all)
        for b in range(3):
            pltpu.async_copy(hs.at[src_all.at[b]], rows[b], gsem[b])

        def outer(t2, carry):
            for b in range(NBUF):
                t = t2 * NBUF + b
                b3 = (b + 3) % NBUF
                pltpu.make_async_copy(hs.at[src_all.at[t]], rows[b],
                                      gsem[b]).wait()
                pltpu.async_copy(rows[b], acc_sh.at[dst_all.at[t]], ssem[b],
                                 add=True)

                @pl.when(t >= 1)
                def _drain():
                    pltpu.make_async_copy(rows[b3], acc_sh.at[dst_all.at[0]],
                                          ssem[b3]).wait()

                @pl.when(t + 3 < nch)
                def _fire():
                    pltpu.async_copy(hs.at[src_all.at[t + 3]], rows[b3],
                                     gsem[b3])
            return carry

        lax.fori_loop(0, ntrip, outer, 0)
        pltpu.make_async_copy(rows[3], acc_sh.at[dst_all.at[0]],
                              ssem[3]).wait()

    plsc.subcore_barrier()
    pltpu.sync_copy(acc_sh.at[pl.ds(sid * RPT, RPT)],
                    out.at[pl.ds(cid * N_PAD + sid * RPT, RPT)])


@functools.partial(
    pl.kernel,
    out_type=jax.ShapeDtypeStruct((UV, D), jnp.float32),
    mesh=_MESH,
    scratch_types=[
        pltpu.VMEM((UV_NCH, CHUNK), jnp.int32),
        pltpu.VMEM((CHUNK, D), jnp.float32),
        pltpu.VMEM((CHUNK, D), jnp.float32),
        pltpu.VMEM((CHUNK, D), jnp.float32),
        pltpu.VMEM((CHUNK, D), jnp.float32),
        pltpu.SemaphoreType.DMA,
        pltpu.SemaphoreType.DMA,
        pltpu.SemaphoreType.DMA,
        pltpu.SemaphoreType.DMA,
        pltpu.SemaphoreType.DMA,
        pltpu.SemaphoreType.DMA,
        pltpu.SemaphoreType.DMA,
        pltpu.SemaphoreType.DMA,
    ],
)
def _sc_gather(h3, idx3, out, idx_all, r0, r1, r2, r3,
               g0, g1, g2, g3, w0, w1, w2, w3):
    rows = (r0, r1, r2, r3)
    gsem = (g0, g1, g2, g3)
    wsem = (w0, w1, w2, w3)
    cid = lax.axis_index("c")
    sid = lax.axis_index("s")
    wid = sid * 2 + cid
    pltpu.sync_copy(idx3.at[wid], idx_all)
    for b in range(3):
        pltpu.async_copy(h3.at[idx_all.at[b]], rows[b], gsem[b])

    def outer(t2, carry):
        for b in range(NBUF):
            t = t2 * NBUF + b
            b3 = (b + 3) % NBUF
            pltpu.make_async_copy(h3.at[idx_all.at[t]], rows[b], gsem[b]).wait()
            pltpu.async_copy(rows[b],
                             out.at[pl.ds(wid * UV_PT + t * CHUNK, CHUNK)],
                             wsem[b])

            @pl.when(t >= 1)
            def _drain():
                pltpu.make_async_copy(
                    rows[b3], out.at[pl.ds(wid * UV_PT, CHUNK)],
                    wsem[b3]).wait()

            @pl.when(t + 3 < UV_NCH)
            def _fire():
                pltpu.async_copy(h3.at[idx_all.at[t + 3]], rows[b3], gsem[b3])
        return carry

    lax.fori_loop(0, UV_NCH // NBUF, outer, 0)
    pltpu.make_async_copy(rows[(UV_NCH - 1) % NBUF],
                          out.at[pl.ds(wid * UV_PT, CHUNK)],
                          wsem[(UV_NCH - 1) % NBUF]).wait()


GB = 8
RB = N_PAD // GB
RB2 = IDX_PAD // GB


def _dinv_col(degp):
    return lax.rsqrt(degp[0] + degp[1] + 1.0)


def _tenc_body(x_ref, ew, eb, h_ref):
    h_ref[...] = (jnp.dot(x_ref[...], ew[...],
                          preferred_element_type=jnp.float32) + eb[...])


def _tenc(x_pad, enc_W, enc_b2):
    return pl.pallas_call(
        _tenc_body,
        grid=(GB,),
        in_specs=[
            pl.BlockSpec((RB, D), lambda i: (i, 0)),
            pl.BlockSpec((D, D), lambda i: (0, 0)),
            pl.BlockSpec((1, D), lambda i: (0, 0)),
        ],
        out_specs=pl.BlockSpec((RB, D), lambda i: (i, 0)),
        out_shape=jax.ShapeDtypeStruct((N_PAD, D), jnp.float32),
    )(x_pad, enc_W, enc_b2)


def _t0_body(h_ref, w0, degp, hs_ref):
    dinv = _dinv_col(degp)
    hs_ref[...] = dinv * jnp.dot(h_ref[...], w0[...],
                                 preferred_element_type=jnp.float32)


def _t0(h, W0, degp):
    return pl.pallas_call(
        _t0_body,
        grid=(GB,),
        in_specs=[
            pl.BlockSpec((RB, D), lambda i: (i, 0)),
            pl.BlockSpec((D, D), lambda i: (0, 0)),
            pl.BlockSpec((2, RB, 1), lambda i: (0, i, 0)),
        ],
        out_specs=pl.BlockSpec((RB, D), lambda i: (i, 0)),
        out_shape=jax.ShapeDtypeStruct((N_PAD, D), jnp.float32),
    )(h, W0, degp)


def _layer_math(sp_ref, hs_ref, h_ref, degp, cb, g, b, m, v):
    dinv = _dinv_col(degp)
    S = sp_ref[0] + sp_ref[1]
    pre = dinv * (S + hs_ref[...]) + cb[...]
    inv_std = lax.rsqrt(v[...] + 1e-5)
    bn = (pre - m[...]) * inv_std * g[...] + b[...]
    return jnp.maximum(bn, 0.0) + h_ref[...], dinv


def _tl_body(sp_ref, hs_ref, h_ref, degp, cb, g, b, m, v, wn, hn_ref, hsn_ref):
    hn, dinv = _layer_math(sp_ref, hs_ref, h_ref, degp, cb, g, b, m, v)
    hn_ref[...] = hn
    hsn_ref[...] = dinv * jnp.dot(hn, wn[...],
                                  preferred_element_type=jnp.float32)


def _tl_last_body(sp_ref, hs_ref, h_ref, degp, cb, g, b, m, v, hn_ref):
    hn, _ = _layer_math(sp_ref, hs_ref, h_ref, degp, cb, g, b, m, v)
    hn_ref[...] = hn


_VEC_SPEC = pl.BlockSpec((1, D), lambda i: (0, 0))


def _tl(Sp, hs, h, degp, cb, g, b, m, v, Wn):
    return pl.pallas_call(
        _tl_body,
        grid=(GB,),
        in_specs=[
            pl.BlockSpec((2, RB, D), lambda i: (0, i, 0)),
            pl.BlockSpec((RB, D), lambda i: (i, 0)),
            pl.BlockSpec((RB, D), lambda i: (i, 0)),
            pl.BlockSpec((2, RB, 1), lambda i: (0, i, 0)),
            _VEC_SPEC, _VEC_SPEC, _VEC_SPEC, _VEC_SPEC, _VEC_SPEC,
            pl.BlockSpec((D, D), lambda i: (0, 0)),
        ],
        out_specs=[pl.BlockSpec((RB, D), lambda i: (i, 0))] * 2,
        out_shape=[jax.ShapeDtypeStruct((N_PAD, D), jnp.float32)] * 2,
    )(Sp, hs, h, degp, cb, g, b, m, v, Wn)


def _tl_last(Sp, hs, h, degp, cb, g, b, m, v):
    return pl.pallas_call(
        _tl_last_body,
        grid=(GB,),
        in_specs=[
            pl.BlockSpec((2, RB, D), lambda i: (0, i, 0)),
            pl.BlockSpec((RB, D), lambda i: (i, 0)),
            pl.BlockSpec((RB, D), lambda i: (i, 0)),
            pl.BlockSpec((2, RB, 1), lambda i: (0, i, 0)),
            _VEC_SPEC, _VEC_SPEC, _VEC_SPEC, _VEC_SPEC, _VEC_SPEC,
        ],
        out_specs=pl.BlockSpec((RB, D), lambda i: (i, 0)),
        out_shape=jax.ShapeDtypeStruct((N_PAD, D), jnp.float32),
    )(Sp, hs, h, degp, cb, g, b, m, v)


def _mlp_body(nu, nv, w1a, w1b, b1, w2, b2, out_ref):
    hid = (jnp.dot(nu[...], w1a[...], preferred_element_type=jnp.float32)
           + jnp.dot(nv[...], w1b[...], preferred_element_type=jnp.float32)
           + b1[...])
    hid = jnp.maximum(hid, 0.0)
    out_ref[...] = jnp.dot(hid, w2[...], preferred_element_type=jnp.float32) + b2[...]


def _mlp(nu, nv, W1a, W1b, b1, W2, b2):
    return pl.pallas_call(
        _mlp_body,
        grid=(GB,),
        in_specs=[
            pl.BlockSpec((RB2, D), lambda i: (i, 0)),
            pl.BlockSpec((RB2, D), lambda i: (i, 0)),
            pl.BlockSpec((D, D), lambda i: (0, 0)),
            pl.BlockSpec((D, D), lambda i: (0, 0)),
            _VEC_SPEC,
            pl.BlockSpec((D, 1), lambda i: (0, 0)),
            pl.BlockSpec((1, 1), lambda i: (0, 0)),
        ],
        out_specs=pl.BlockSpec((RB2, 1), lambda i: (i, 0)),
        out_shape=jax.ShapeDtypeStruct((IDX_PAD, 1), jnp.float32),
    )(nu, nv, W1a, W1b, b1, W2, b2)


def kernel(x, edge_index, num_graphs, branch_u, branch_v, enc_W, enc_b,
           conv_W, conv_b, bn_gamma, bn_beta, bn_mean, bn_var,
           mlp_W1, mlp_b1, mlp_W2, mlp_b2):
    src = edge_index[0]
    dst = edge_index[1]
    pad_e = E_PAD - E
    srcp = jnp.concatenate([src, jnp.zeros((pad_e,), jnp.int32)])
    dstp = jnp.concatenate([dst, jnp.full((pad_e,), N_NODES, jnp.int32)])
    dstp3 = dstp.reshape(N_TILES, NCH, CHUNK)
    def _split_idx(a):
        e0 = 16 * NPASS * CH_A * SCHUNK
        c0b = a[:e0].reshape(16, NPASS, CH_A, SCHUNK)
        c1b = a[e0:].reshape(16, NPASS, CH_B, SCHUNK)
        c1b = jnp.pad(c1b, ((0, 0), (0, 0), (0, CH_A - CH_B), (0, 0)))
        return jnp.stack([c0b, c1b], axis=1).reshape(
            N_TILES, NPASS, CH_A, SCHUNK)

    srcp4 = _split_idx(srcp)
    dstp4 = _split_idx(dstp)
    x_pad = jnp.pad(x, ((0, N_PAD - N_NODES), (0, 0)))
    zrows = jnp.zeros((RPT, D), jnp.float32)
    onesr = jnp.ones((CHUNK, D), jnp.float32)

    degp = _sc_deg(dstp3, onesr, zrows).reshape(2, N_PAD, D)[:, :, :1]

    h = _tenc(x_pad, enc_W, enc_b.reshape(1, D))
    hs = _t0(h, conv_W[0], degp)
    for i in range(3):
        Sp = _sc_scatter(hs, srcp4, dstp4, zrows).reshape(2, N_PAD, D)
        args = (Sp, hs, h, degp, conv_b[i].reshape(1, D),
                bn_gamma[i].reshape(1, D), bn_beta[i].reshape(1, D),
                bn_mean[i].reshape(1, D), bn_var[i].reshape(1, D))
        if i < 2:
            h, hs = _tl(*args, conv_W[i + 1])
        else:
            h = _tl_last(*args)

    nb = branch_u.shape[0]
    num_graphs_zero = (jnp.asarray(num_graphs) * 0).astype(branch_u.dtype)
    offsets = (jnp.arange(NUM_GRAPHS, dtype=branch_u.dtype) * NODES_PER_GRAPH
               + num_graphs_zero)
    u_idx = (branch_u[None, :] + offsets[:, None]).reshape(-1)
    v_idx = (branch_v[None, :] + offsets[:, None]).reshape(-1)
    nout = NUM_GRAPHS * nb
    pad_i = IDX_PAD - nout
    uv = jnp.concatenate([
        u_idx, jnp.zeros((pad_i,), branch_u.dtype),
        v_idx, jnp.zeros((pad_i,), branch_u.dtype),
    ]).reshape(N_TILES, UV_NCH, CHUNK)
    gth = _sc_gather(h, uv)
    out_full = _mlp(gth[:IDX_PAD], gth[IDX_PAD:], mlp_W1[:D], mlp_W1[D:],
                    mlp_b1.reshape(1, D), mlp_W2, mlp_b2.reshape(1, 1))
    return out_full[:nout]


# speedup vs baseline: 1.0515x; 1.0515x over previous
import functools

import jax
import jax.numpy as jnp
from jax import lax
from jax.experimental import pallas as pl
from jax.experimental.pallas import tpu as pltpu
from jax.experimental.pallas import tpu_sc as plsc

N_NODES = 10020
D = 128
N_PAD = 10240
N_TILES = 32
RPT = N_PAD // 16
E = 320640
CHUNK = 128
NBUF = 4
_NCH_MIN = -(-E // (N_TILES * CHUNK))
NCH = -(-_NCH_MIN // NBUF) * NBUF
E_PAD = N_TILES * CHUNK * NCH
EPT = E_PAD // N_TILES
SCHUNK = 128
NPASS = 2
QTR = EPT // SCHUNK // NPASS
NUM_GRAPHS = 334
NODES_PER_GRAPH = 30
IDX_PAD = 16384
UV = 2 * IDX_PAD
UV_PT = UV // N_TILES
UV_NCH = UV_PT // CHUNK

_MESH = plsc.VectorSubcoreMesh(core_axis_name="c", subcore_axis_name="s")


@functools.partial(
    pl.kernel,
    out_type=jax.ShapeDtypeStruct((2 * N_PAD, D), jnp.float32),
    mesh=_MESH,
    scratch_types=[
        pltpu.VMEM((NCH, CHUNK), jnp.int32),
        pltpu.VMEM((CHUNK, D), jnp.float32),
        pltpu.VMEM_SHARED((N_PAD, D), jnp.float32),
        pltpu.SemaphoreType.DMA,
    ],
)
def _sc_deg(dstp3, onesr, zrows, out, dst_all, ones_v, acc_sh, ssem):
    cid = lax.axis_index("c")
    sid = lax.axis_index("s")
    wid = sid * 2 + cid
    pltpu.sync_copy(zrows, acc_sh.at[pl.ds(sid * RPT, RPT)])
    pltpu.sync_copy(dstp3.at[wid], dst_all)
    pltpu.sync_copy(onesr, ones_v)
    plsc.subcore_barrier()

    def fire(t, carry):
        pltpu.async_copy(ones_v, acc_sh.at[dst_all.at[t]], ssem, add=True)
        return carry

    lax.fori_loop(0, NCH, fire, 0)

    def drain(t, carry):
        pltpu.make_async_copy(ones_v, acc_sh.at[dst_all.at[0]], ssem).wait()
        return carry

    lax.fori_loop(0, NCH, drain, 0)
    plsc.subcore_barrier()
    pltpu.sync_copy(acc_sh.at[pl.ds(sid * RPT, RPT)],
                    out.at[pl.ds(cid * N_PAD + sid * RPT, RPT)])


@functools.partial(
    pl.kernel,
    out_type=jax.ShapeDtypeStruct((2 * N_PAD, D), jnp.float32),
    mesh=_MESH,
    scratch_types=[
        pltpu.VMEM((QTR, SCHUNK), jnp.int32),
        pltpu.VMEM((QTR, SCHUNK), jnp.int32),
        pltpu.VMEM((SCHUNK, D), jnp.float32),
        pltpu.VMEM((SCHUNK, D), jnp.float32),
        pltpu.VMEM_SHARED((N_PAD, D), jnp.float32),
        pltpu.SemaphoreType.DMA,
        pltpu.SemaphoreType.DMA,
    ],
)
def _sc_scatter(hs, srcp4, dstp4, zrows, out, src_all, dst_all,
                r0, r1, acc_sh, g0, g1):
    rows = (r0, r1)
    gsem = (g0, g1)
    cid = lax.axis_index("c")
    sid = lax.axis_index("s")
    wid = sid * 2 + cid
    pltpu.sync_copy(zrows, acc_sh.at[pl.ds(sid * RPT, RPT)])
    plsc.subcore_barrier()

    for p in range(NPASS):
        pltpu.sync_copy(srcp4.at[wid, p], src_all)
        pltpu.sync_copy(dstp4.at[wid, p], dst_all)
        pltpu.async_copy(hs.at[src_all.at[0]], rows[0], gsem[0])

        def outer(t2, carry):
            for b in range(2):
                t = t2 * 2 + b
                ob = 1 - b
                pltpu.make_async_copy(hs.at[src_all.at[t]], rows[b],
                                      gsem[b]).wait()

                @pl.when(t + 1 < QTR)
                def _fire():
                    pltpu.async_copy(hs.at[src_all.at[t + 1]], rows[ob],
                                     gsem[ob])

                pltpu.sync_copy(rows[b], acc_sh.at[dst_all.at[t]], add=True)
            return carry

        lax.fori_loop(0, QTR // 2, outer, 0)

    plsc.subcore_barrier()
    pltpu.sync_copy(acc_sh.at[pl.ds(sid * RPT, RPT)],
                    out.at[pl.ds(cid * N_PAD + sid * RPT, RPT)])


@functools.partial(
    pl.kernel,
    out_type=jax.ShapeDtypeStruct((UV, D), jnp.float32),
    mesh=_MESH,
    scratch_types=[
        pltpu.VMEM((UV_NCH, CHUNK), jnp.int32),
        pltpu.VMEM((CHUNK, D), jnp.float32),
        pltpu.VMEM((CHUNK, D), jnp.float32),
        pltpu.VMEM((CHUNK, D), jnp.float32),
        pltpu.VMEM((CHUNK, D), jnp.float32),
        pltpu.SemaphoreType.DMA,
        pltpu.SemaphoreType.DMA,
        pltpu.SemaphoreType.DMA,
        pltpu.SemaphoreType.DMA,
        pltpu.SemaphoreType.DMA,
        pltpu.SemaphoreType.DMA,
        pltpu.SemaphoreType.DMA,
        pltpu.SemaphoreType.DMA,
    ],
)
def _sc_gather(h3, idx3, out, idx_all, r0, r1, r2, r3,
               g0, g1, g2, g3, w0, w1, w2, w3):
    rows = (r0, r1, r2, r3)
    gsem = (g0, g1, g2, g3)
    wsem = (w0, w1, w2, w3)
    cid = lax.axis_index("c")
    sid = lax.axis_index("s")
    wid = sid * 2 + cid
    pltpu.sync_copy(idx3.at[wid], idx_all)
    for b in range(3):
        pltpu.async_copy(h3.at[idx_all.at[b]], rows[b], gsem[b])

    def outer(t2, carry):
        for b in range(NBUF):
            t = t2 * NBUF + b
            b3 = (b + 3) % NBUF
            pltpu.make_async_copy(h3.at[idx_all.at[t]], rows[b], gsem[b]).wait()
            pltpu.async_copy(rows[b],
                             out.at[pl.ds(wid * UV_PT + t * CHUNK, CHUNK)],
                             wsem[b])

            @pl.when(t >= 1)
            def _drain():
                pltpu.make_async_copy(
                    rows[b3], out.at[pl.ds(wid * UV_PT, CHUNK)],
                    wsem[b3]).wait()

            @pl.when(t + 3 < UV_NCH)
            def _fire():
                pltpu.async_copy(h3.at[idx_all.at[t + 3]], rows[b3], gsem[b3])
        return carry

    lax.fori_loop(0, UV_NCH // NBUF, outer, 0)
    pltpu.make_async_copy(rows[(UV_NCH - 1) % NBUF],
                          out.at[pl.ds(wid * UV_PT, CHUNK)],
                          wsem[(UV_NCH - 1) % NBUF]).wait()


GB = 8
RB = N_PAD // GB
RB2 = IDX_PAD // GB


def _dinv_col(degp):
    return lax.rsqrt(degp[0] + degp[1] + 1.0)


def _tenc_body(x_ref, ew, eb, h_ref):
    h_ref[...] = (jnp.dot(x_ref[...], ew[...],
                          preferred_element_type=jnp.float32) + eb[...])


def _tenc(x_pad, enc_W, enc_b2):
    return pl.pallas_call(
        _tenc_body,
        grid=(GB,),
        in_specs=[
            pl.BlockSpec((RB, D), lambda i: (i, 0)),
            pl.BlockSpec((D, D), lambda i: (0, 0)),
            pl.BlockSpec((1, D), lambda i: (0, 0)),
        ],
        out_specs=pl.BlockSpec((RB, D), lambda i: (i, 0)),
        out_shape=jax.ShapeDtypeStruct((N_PAD, D), jnp.float32),
    )(x_pad, enc_W, enc_b2)


def _t0_body(h_ref, w0, degp, hs_ref):
    dinv = _dinv_col(degp)
    hs_ref[...] = dinv * jnp.dot(h_ref[...], w0[...],
                                 preferred_element_type=jnp.float32)


def _t0(h, W0, degp):
    return pl.pallas_call(
        _t0_body,
        grid=(GB,),
        in_specs=[
            pl.BlockSpec((RB, D), lambda i: (i, 0)),
            pl.BlockSpec((D, D), lambda i: (0, 0)),
            pl.BlockSpec((2, RB, 1), lambda i: (0, i, 0)),
        ],
        out_specs=pl.BlockSpec((RB, D), lambda i: (i, 0)),
        out_shape=jax.ShapeDtypeStruct((N_PAD, D), jnp.float32),
    )(h, W0, degp)


def _layer_math(sp_ref, hs_ref, h_ref, degp, cb, g, b, m, v):
    dinv = _dinv_col(degp)
    S = sp_ref[0] + sp_ref[1]
    pre = dinv * (S + hs_ref[...]) + cb[...]
    inv_std = lax.rsqrt(v[...] + 1e-5)
    bn = (pre - m[...]) * inv_std * g[...] + b[...]
    return jnp.maximum(bn, 0.0) + h_ref[...], dinv


def _tl_body(sp_ref, hs_ref, h_ref, degp, cb, g, b, m, v, wn, hn_ref, hsn_ref):
    hn, dinv = _layer_math(sp_ref, hs_ref, h_ref, degp, cb, g, b, m, v)
    hn_ref[...] = hn
    hsn_ref[...] = dinv * jnp.dot(hn, wn[...],
                                  preferred_element_type=jnp.float32)


def _tl_last_body(sp_ref, hs_ref, h_ref, degp, cb, g, b, m, v, hn_ref):
    hn, _ = _layer_math(sp_ref, hs_ref, h_ref, degp, cb, g, b, m, v)
    hn_ref[...] = hn


_VEC_SPEC = pl.BlockSpec((1, D), lambda i: (0, 0))


def _tl(Sp, hs, h, degp, cb, g, b, m, v, Wn):
    return pl.pallas_call(
        _tl_body,
        grid=(GB,),
        in_specs=[
            pl.BlockSpec((2, RB, D), lambda i: (0, i, 0)),
            pl.BlockSpec((RB, D), lambda i: (i, 0)),
            pl.BlockSpec((RB, D), lambda i: (i, 0)),
            pl.BlockSpec((2, RB, 1), lambda i: (0, i, 0)),
            _VEC_SPEC, _VEC_SPEC, _VEC_SPEC, _VEC_SPEC, _VEC_SPEC,
            pl.BlockSpec((D, D), lambda i: (0, 0)),
        ],
        out_specs=[pl.BlockSpec((RB, D), lambda i: (i, 0))] * 2,
        out_shape=[jax.ShapeDtypeStruct((N_PAD, D), jnp.float32)] * 2,
    )(Sp, hs, h, degp, cb, g, b, m, v, Wn)


def _tl_last(Sp, hs, h, degp, cb, g, b, m, v):
    return pl.pallas_call(
        _tl_last_body,
        grid=(GB,),
        in_specs=[
            pl.BlockSpec((2, RB, D), lambda i: (0, i, 0)),
            pl.BlockSpec((RB, D), lambda i: (i, 0)),
            pl.BlockSpec((RB, D), lambda i: (i, 0)),
            pl.BlockSpec((2, RB, 1), lambda i: (0, i, 0)),
            _VEC_SPEC, _VEC_SPEC, _VEC_SPEC, _VEC_SPEC, _VEC_SPEC,
        ],
        out_specs=pl.BlockSpec((RB, D), lambda i: (i, 0)),
        out_shape=jax.ShapeDtypeStruct((N_PAD, D), jnp.float32),
    )(Sp, hs, h, degp, cb, g, b, m, v)


def _mlp_body(nu, nv, w1a, w1b, b1, w2, b2, out_ref):
    hid = (jnp.dot(nu[...], w1a[...], preferred_element_type=jnp.float32)
           + jnp.dot(nv[...], w1b[...], preferred_element_type=jnp.float32)
           + b1[...])
    hid = jnp.maximum(hid, 0.0)
    out_ref[...] = jnp.dot(hid, w2[...], preferred_element_type=jnp.float32) + b2[...]


def _mlp(nu, nv, W1a, W1b, b1, W2, b2):
    return pl.pallas_call(
        _mlp_body,
        grid=(GB,),
        in_specs=[
            pl.BlockSpec((RB2, D), lambda i: (i, 0)),
            pl.BlockSpec((RB2, D), lambda i: (i, 0)),
            pl.BlockSpec((D, D), lambda i: (0, 0)),
            pl.BlockSpec((D, D), lambda i: (0, 0)),
            _VEC_SPEC,
            pl.BlockSpec((D, 1), lambda i: (0, 0)),
            pl.BlockSpec((1, 1), lambda i: (0, 0)),
        ],
        out_specs=pl.BlockSpec((RB2, 1), lambda i: (i, 0)),
        out_shape=jax.ShapeDtypeStruct((IDX_PAD, 1), jnp.float32),
    )(nu, nv, W1a, W1b, b1, W2, b2)


def kernel(x, edge_index, num_graphs, branch_u, branch_v, enc_W, enc_b,
           conv_W, conv_b, bn_gamma, bn_beta, bn_mean, bn_var,
           mlp_W1, mlp_b1, mlp_W2, mlp_b2):
    src = edge_index[0]
    dst = edge_index[1]
    pad_e = E_PAD - E
    srcp = jnp.concatenate([src, jnp.zeros((pad_e,), jnp.int32)])
    dstp = jnp.concatenate([dst, jnp.full((pad_e,), N_NODES, jnp.int32)])
    dstp3 = dstp.reshape(N_TILES, NCH, CHUNK)
    srcp4 = srcp.reshape(N_TILES, NPASS, QTR, SCHUNK)
    dstp4 = dstp.reshape(N_TILES, NPASS, QTR, SCHUNK)
    x_pad = jnp.pad(x, ((0, N_PAD - N_NODES), (0, 0)))
    zrows = jnp.zeros((RPT, D), jnp.float32)
    onesr = jnp.ones((CHUNK, D), jnp.float32)

    degp = _sc_deg(dstp3, onesr, zrows).reshape(2, N_PAD, D)[:, :, :1]

    h = _tenc(x_pad, enc_W, enc_b.reshape(1, D))
    hs = _t0(h, conv_W[0], degp)
    for i in range(3):
        Sp = _sc_scatter(hs, srcp4, dstp4, zrows).reshape(2, N_PAD, D)
        args = (Sp, hs, h, degp, conv_b[i].reshape(1, D),
                bn_gamma[i].reshape(1, D), bn_beta[i].reshape(1, D),
                bn_mean[i].reshape(1, D), bn_var[i].reshape(1, D))
        if i < 2:
            h, hs = _tl(*args, conv_W[i + 1])
        else:
            h = _tl_last(*args)

    nb = branch_u.shape[0]
    num_graphs_zero = (jnp.asarray(num_graphs) * 0).astype(branch_u.dtype)
    offsets = (jnp.arange(NUM_GRAPHS, dtype=branch_u.dtype) * NODES_PER_GRAPH
               + num_graphs_zero)
    u_idx = (branch_u[None, :] + offsets[:, None]).reshape(-1)
    v_idx = (branch_v[None, :] + offsets[:, None]).reshape(-1)
    nout = NUM_GRAPHS * nb
    pad_i = IDX_PAD - nout
    uv = jnp.concatenate([
        u_idx, jnp.zeros((pad_i,), branch_u.dtype),
        v_idx, jnp.zeros((pad_i,), branch_u.dtype),
    ]).reshape(N_TILES, UV_NCH, CHUNK)
    gth = _sc_gather(h, uv)
    out_full = _mlp(gth[:IDX_PAD], gth[IDX_PAD:], mlp_W1[:D], mlp_W1[D:],
                    mlp_b1.reshape(1, D), mlp_W2, mlp_b2.reshape(1, 1))
    return out_full[:nout]


# speedup vs baseline: 1.0567x; 1.0049x over previous
import functools

import jax
import jax.numpy as jnp
from jax import lax
from jax.experimental import pallas as pl
from jax.experimental.pallas import tpu as pltpu
from jax.experimental.pallas import tpu_sc as plsc

N_NODES = 10020
D = 128
N_PAD = 10240
N_TILES = 32
RPT = N_PAD // 16
E = 320640
CHUNK = 128
NBUF = 4
_NCH_MIN = -(-E // (N_TILES * CHUNK))
NCH = -(-_NCH_MIN // NBUF) * NBUF
E_PAD = N_TILES * CHUNK * NCH
EPT = E_PAD // N_TILES
SCHUNK = 128
NPASS = 2
CP_A = 56
CP_B = 24
_TOT = 16 * NPASS * (CP_A + CP_B) * SCHUNK
assert _TOT == E_PAD, (_TOT, E_PAD)
NUM_GRAPHS = 334
NODES_PER_GRAPH = 30
IDX_PAD = 16384
UV = 2 * IDX_PAD
UV_PT = UV // N_TILES
UV_NCH = UV_PT // CHUNK

_MESH = plsc.VectorSubcoreMesh(core_axis_name="c", subcore_axis_name="s")


@functools.partial(
    pl.kernel,
    out_type=jax.ShapeDtypeStruct((2 * N_PAD, D), jnp.float32),
    mesh=_MESH,
    scratch_types=[
        pltpu.VMEM((NCH, CHUNK), jnp.int32),
        pltpu.VMEM((CHUNK, D), jnp.float32),
        pltpu.VMEM_SHARED((N_PAD, D), jnp.float32),
        pltpu.SemaphoreType.DMA,
    ],
)
def _sc_deg(dstp3, onesr, zrows, out, dst_all, ones_v, acc_sh, ssem):
    cid = lax.axis_index("c")
    sid = lax.axis_index("s")
    wid = sid * 2 + cid
    pltpu.sync_copy(zrows, acc_sh.at[pl.ds(sid * RPT, RPT)])
    pltpu.sync_copy(dstp3.at[wid], dst_all)
    pltpu.sync_copy(onesr, ones_v)
    plsc.subcore_barrier()

    def fire(t, carry):
        pltpu.async_copy(ones_v, acc_sh.at[dst_all.at[t]], ssem, add=True)
        return carry

    lax.fori_loop(0, NCH, fire, 0)

    def drain(t, carry):
        pltpu.make_async_copy(ones_v, acc_sh.at[dst_all.at[0]], ssem).wait()
        return carry

    lax.fori_loop(0, NCH, drain, 0)
    plsc.subcore_barrier()
    pltpu.sync_copy(acc_sh.at[pl.ds(sid * RPT, RPT)],
                    out.at[pl.ds(cid * N_PAD + sid * RPT, RPT)])


@functools.partial(
    pl.kernel,
    out_type=jax.ShapeDtypeStruct((2 * N_PAD, D), jnp.float32),
    mesh=_MESH,
    scratch_types=[
        pltpu.VMEM((CP_A, SCHUNK), jnp.int32),
        pltpu.VMEM((CP_A, SCHUNK), jnp.int32),
        pltpu.VMEM((SCHUNK, D), jnp.float32),
        pltpu.VMEM((SCHUNK, D), jnp.float32),
        pltpu.VMEM_SHARED((N_PAD, D), jnp.float32),
        pltpu.SemaphoreType.DMA,
        pltpu.SemaphoreType.DMA,
    ],
)
def _sc_scatter(hs, srcp4, dstp4, zrows, out, src_all, dst_all,
                r0, r1, acc_sh, g0, g1):
    rows = (r0, r1)
    gsem = (g0, g1)
    cid = lax.axis_index("c")
    sid = lax.axis_index("s")
    wid = sid * 2 + cid
    pltpu.sync_copy(zrows, acc_sh.at[pl.ds(sid * RPT, RPT)])
    plsc.subcore_barrier()

    for p in range(NPASS):
        pltpu.sync_copy(srcp4.at[wid, p], src_all)
        pltpu.sync_copy(dstp4.at[wid, p], dst_all)

        @pl.when(cid == 0)
        def _pipelined():
            pltpu.async_copy(hs.at[src_all.at[0]], rows[0], gsem[0])

            def outer(t2, carry):
                for b in range(2):
                    t = t2 * 2 + b
                    ob = 1 - b
                    pltpu.make_async_copy(hs.at[src_all.at[t]], rows[b],
                                          gsem[b]).wait()

                    @pl.when(t + 1 < CP_A)
                    def _fire():
                        pltpu.async_copy(hs.at[src_all.at[t + 1]], rows[ob],
                                         gsem[ob])

                    pltpu.sync_copy(rows[b], acc_sh.at[dst_all.at[t]],
                                    add=True)
                return carry

            lax.fori_loop(0, CP_A // 2, outer, 0)

        @pl.when(cid == 1)
        def _serial():
            def sbody(t, carry):
                pltpu.async_copy(hs.at[src_all.at[t]], rows[0], gsem[0]).wait()
                pltpu.sync_copy(rows[0], acc_sh.at[dst_all.at[t]], add=True)
                return carry

            lax.fori_loop(0, CP_B, sbody, 0)

    plsc.subcore_barrier()
    pltpu.sync_copy(acc_sh.at[pl.ds(sid * RPT, RPT)],
                    out.at[pl.ds(cid * N_PAD + sid * RPT, RPT)])


@functools.partial(
    pl.kernel,
    out_type=jax.ShapeDtypeStruct((UV, D), jnp.float32),
    mesh=_MESH,
    scratch_types=[
        pltpu.VMEM((UV_NCH, CHUNK), jnp.int32),
        pltpu.VMEM((CHUNK, D), jnp.float32),
        pltpu.VMEM((CHUNK, D), jnp.float32),
        pltpu.VMEM((CHUNK, D), jnp.float32),
        pltpu.VMEM((CHUNK, D), jnp.float32),
        pltpu.SemaphoreType.DMA,
        pltpu.SemaphoreType.DMA,
        pltpu.SemaphoreType.DMA,
        pltpu.SemaphoreType.DMA,
        pltpu.SemaphoreType.DMA,
        pltpu.SemaphoreType.DMA,
        pltpu.SemaphoreType.DMA,
        pltpu.SemaphoreType.DMA,
    ],
)
def _sc_gather(h3, idx3, out, idx_all, r0, r1, r2, r3,
               g0, g1, g2, g3, w0, w1, w2, w3):
    rows = (r0, r1, r2, r3)
    gsem = (g0, g1, g2, g3)
    wsem = (w0, w1, w2, w3)
    cid = lax.axis_index("c")
    sid = lax.axis_index("s")
    wid = sid * 2 + cid
    pltpu.sync_copy(idx3.at[wid], idx_all)
    for b in range(3):
        pltpu.async_copy(h3.at[idx_all.at[b]], rows[b], gsem[b])

    def outer(t2, carry):
        for b in range(NBUF):
            t = t2 * NBUF + b
            b3 = (b + 3) % NBUF
            pltpu.make_async_copy(h3.at[idx_all.at[t]], rows[b], gsem[b]).wait()
            pltpu.async_copy(rows[b],
                             out.at[pl.ds(wid * UV_PT + t * CHUNK, CHUNK)],
                             wsem[b])

            @pl.when(t >= 1)
            def _drain():
                pltpu.make_async_copy(
                    rows[b3], out.at[pl.ds(wid * UV_PT, CHUNK)],
                    wsem[b3]).wait()

            @pl.when(t + 3 < UV_NCH)
            def _fire():
                pltpu.async_copy(h3.at[idx_all.at[t + 3]], rows[b3], gsem[b3])
        return carry

    lax.fori_loop(0, UV_NCH // NBUF, outer, 0)
    pltpu.make_async_copy(rows[(UV_NCH - 1) % NBUF],
                          out.at[pl.ds(wid * UV_PT, CHUNK)],
                          wsem[(UV_NCH - 1) % NBUF]).wait()


GB = 8
RB = N_PAD // GB
RB2 = IDX_PAD // GB


def _dinv_col(degp):
    return lax.rsqrt(degp[0] + degp[1] + 1.0)


def _tenc_body(x_ref, ew, eb, h_ref):
    h_ref[...] = (jnp.dot(x_ref[...], ew[...],
                          preferred_element_type=jnp.float32) + eb[...])


def _tenc(x_pad, enc_W, enc_b2):
    return pl.pallas_call(
        _tenc_body,
        grid=(GB,),
        in_specs=[
            pl.BlockSpec((RB, D), lambda i: (i, 0)),
            pl.BlockSpec((D, D), lambda i: (0, 0)),
            pl.BlockSpec((1, D), lambda i: (0, 0)),
        ],
        out_specs=pl.BlockSpec((RB, D), lambda i: (i, 0)),
        out_shape=jax.ShapeDtypeStruct((N_PAD, D), jnp.float32),
    )(x_pad, enc_W, enc_b2)


def _t0_body(h_ref, w0, degp, hs_ref):
    dinv = _dinv_col(degp)
    hs_ref[...] = dinv * jnp.dot(h_ref[...], w0[...],
                                 preferred_element_type=jnp.float32)


def _t0(h, W0, degp):
    return pl.pallas_call(
        _t0_body,
        grid=(GB,),
        in_specs=[
            pl.BlockSpec((RB, D), lambda i: (i, 0)),
            pl.BlockSpec((D, D), lambda i: (0, 0)),
            pl.BlockSpec((2, RB, 1), lambda i: (0, i, 0)),
        ],
        out_specs=pl.BlockSpec((RB, D), lambda i: (i, 0)),
        out_shape=jax.ShapeDtypeStruct((N_PAD, D), jnp.float32),
    )(h, W0, degp)


def _layer_math(sp_ref, hs_ref, h_ref, degp, cb, g, b, m, v):
    dinv = _dinv_col(degp)
    S = sp_ref[0] + sp_ref[1]
    pre = dinv * (S + hs_ref[...]) + cb[...]
    inv_std = lax.rsqrt(v[...] + 1e-5)
    bn = (pre - m[...]) * inv_std * g[...] + b[...]
    return jnp.maximum(bn, 0.0) + h_ref[...], dinv


def _tl_body(sp_ref, hs_ref, h_ref, degp, cb, g, b, m, v, wn, hn_ref, hsn_ref):
    hn, dinv = _layer_math(sp_ref, hs_ref, h_ref, degp, cb, g, b, m, v)
    hn_ref[...] = hn
    hsn_ref[...] = dinv * jnp.dot(hn, wn[...],
                                  preferred_element_type=jnp.float32)


def _tl_last_body(sp_ref, hs_ref, h_ref, degp, cb, g, b, m, v, hn_ref):
    hn, _ = _layer_math(sp_ref, hs_ref, h_ref, degp, cb, g, b, m, v)
    hn_ref[...] = hn


_VEC_SPEC = pl.BlockSpec((1, D), lambda i: (0, 0))


def _tl(Sp, hs, h, degp, cb, g, b, m, v, Wn):
    return pl.pallas_call(
        _tl_body,
        grid=(GB,),
        in_specs=[
            pl.BlockSpec((2, RB, D), lambda i: (0, i, 0)),
            pl.BlockSpec((RB, D), lambda i: (i, 0)),
            pl.BlockSpec((RB, D), lambda i: (i, 0)),
            pl.BlockSpec((2, RB, 1), lambda i: (0, i, 0)),
            _VEC_SPEC, _VEC_SPEC, _VEC_SPEC, _VEC_SPEC, _VEC_SPEC,
            pl.BlockSpec((D, D), lambda i: (0, 0)),
        ],
        out_specs=[pl.BlockSpec((RB, D), lambda i: (i, 0))] * 2,
        out_shape=[jax.ShapeDtypeStruct((N_PAD, D), jnp.float32)] * 2,
    )(Sp, hs, h, degp, cb, g, b, m, v, Wn)


def _tl_last(Sp, hs, h, degp, cb, g, b, m, v):
    return pl.pallas_call(
        _tl_last_body,
        grid=(GB,),
        in_specs=[
            pl.BlockSpec((2, RB, D), lambda i: (0, i, 0)),
            pl.BlockSpec((RB, D), lambda i: (i, 0)),
            pl.BlockSpec((RB, D), lambda i: (i, 0)),
            pl.BlockSpec((2, RB, 1), lambda i: (0, i, 0)),
            _VEC_SPEC, _VEC_SPEC, _VEC_SPEC, _VEC_SPEC, _VEC_SPEC,
        ],
        out_specs=pl.BlockSpec((RB, D), lambda i: (i, 0)),
        out_shape=jax.ShapeDtypeStruct((N_PAD, D), jnp.float32),
    )(Sp, hs, h, degp, cb, g, b, m, v)


def _mlp_body(nu, nv, w1a, w1b, b1, w2, b2, out_ref):
    hid = (jnp.dot(nu[...], w1a[...], preferred_element_type=jnp.float32)
           + jnp.dot(nv[...], w1b[...], preferred_element_type=jnp.float32)
           + b1[...])
    hid = jnp.maximum(hid, 0.0)
    out_ref[...] = jnp.dot(hid, w2[...], preferred_element_type=jnp.float32) + b2[...]


def _mlp(nu, nv, W1a, W1b, b1, W2, b2):
    return pl.pallas_call(
        _mlp_body,
        grid=(GB,),
        in_specs=[
            pl.BlockSpec((RB2, D), lambda i: (i, 0)),
            pl.BlockSpec((RB2, D), lambda i: (i, 0)),
            pl.BlockSpec((D, D), lambda i: (0, 0)),
            pl.BlockSpec((D, D), lambda i: (0, 0)),
            _VEC_SPEC,
            pl.BlockSpec((D, 1), lambda i: (0, 0)),
            pl.BlockSpec((1, 1), lambda i: (0, 0)),
        ],
        out_specs=pl.BlockSpec((RB2, 1), lambda i: (i, 0)),
        out_shape=jax.ShapeDtypeStruct((IDX_PAD, 1), jnp.float32),
    )(nu, nv, W1a, W1b, b1, W2, b2)


def kernel(x, edge_index, num_graphs, branch_u, branch_v, enc_W, enc_b,
           conv_W, conv_b, bn_gamma, bn_beta, bn_mean, bn_var,
           mlp_W1, mlp_b1, mlp_W2, mlp_b2):
    src = edge_index[0]
    dst = edge_index[1]
    pad_e = E_PAD - E
    srcp = jnp.concatenate([src, jnp.zeros((pad_e,), jnp.int32)])
    dstp = jnp.concatenate([dst, jnp.full((pad_e,), N_NODES, jnp.int32)])
    dstp3 = dstp.reshape(N_TILES, NCH, CHUNK)
    def _split_idx(a):
        e0 = 16 * NPASS * CP_A * SCHUNK
        c0b = a[:e0].reshape(16, NPASS, CP_A, SCHUNK)
        c1b = a[e0:].reshape(16, NPASS, CP_B, SCHUNK)
        c1b = jnp.pad(c1b, ((0, 0), (0, 0), (0, CP_A - CP_B), (0, 0)))
        return jnp.stack([c0b, c1b], axis=1).reshape(
            N_TILES, NPASS, CP_A, SCHUNK)

    srcp4 = _split_idx(srcp)
    dstp4 = _split_idx(dstp)
    x_pad = jnp.pad(x, ((0, N_PAD - N_NODES), (0, 0)))
    zrows = jnp.zeros((RPT, D), jnp.float32)
    onesr = jnp.ones((CHUNK, D), jnp.float32)

    degp = _sc_deg(dstp3, onesr, zrows).reshape(2, N_PAD, D)[:, :, :1]

    h = _tenc(x_pad, enc_W, enc_b.reshape(1, D))
    hs = _t0(h, conv_W[0], degp)
    for i in range(3):
        Sp = _sc_scatter(hs, srcp4, dstp4, zrows).reshape(2, N_PAD, D)
        args = (Sp, hs, h, degp, conv_b[i].reshape(1, D),
                bn_gamma[i].reshape(1, D), bn_beta[i].reshape(1, D),
                bn_mean[i].reshape(1, D), bn_var[i].reshape(1, D))
        if i < 2:
            h, hs = _tl(*args, conv_W[i + 1])
        else:
            h = _tl_last(*args)

    nb = branch_u.shape[0]
    num_graphs_zero = (jnp.asarray(num_graphs) * 0).astype(branch_u.dtype)
    offsets = (jnp.arange(NUM_GRAPHS, dtype=branch_u.dtype) * NODES_PER_GRAPH
               + num_graphs_zero)
    u_idx = (branch_u[None, :] + offsets[:, None]).reshape(-1)
    v_idx = (branch_v[None, :] + offsets[:, None]).reshape(-1)
    nout = NUM_GRAPHS * nb
    pad_i = IDX_PAD - nout
    uv = jnp.concatenate([
        u_idx, jnp.zeros((pad_i,), branch_u.dtype),
        v_idx, jnp.zeros((pad_i,), branch_u.dtype),
    ]).reshape(N_TILES, UV_NCH, CHUNK)
    gth = _sc_gather(h, uv)
    out_full = _mlp(gth[:IDX_PAD], gth[IDX_PAD:], mlp_W1[:D], mlp_W1[D:],
                    mlp_b1.reshape(1, D), mlp_W2, mlp_b2.reshape(1, 1))
    return out_full[:nout]


# speedup vs baseline: 1.1131x; 1.0534x over previous
import functools

import jax
import jax.numpy as jnp
from jax import lax
from jax.experimental import pallas as pl
from jax.experimental.pallas import tpu as pltpu
from jax.experimental.pallas import tpu_sc as plsc

N_NODES = 10020
D = 128
N_PAD = 10240
N_TILES = 32
RPT = N_PAD // 16
E = 320640
CHUNK = 128
NBUF = 4
_NCH_MIN = -(-E // (N_TILES * CHUNK))
NCH = -(-_NCH_MIN // NBUF) * NBUF
E_PAD = N_TILES * CHUNK * NCH
EPT = E_PAD // N_TILES
SCHUNK = 128
NPASS = 2
CP_A = 60
CP_B = 20
_TOT = 16 * NPASS * (CP_A + CP_B) * SCHUNK
assert _TOT == E_PAD, (_TOT, E_PAD)
NUM_GRAPHS = 334
NODES_PER_GRAPH = 30
IDX_PAD = 16384
UV = 2 * IDX_PAD
UV_PT = UV // N_TILES
UV_NCH = UV_PT // CHUNK

_MESH = plsc.VectorSubcoreMesh(core_axis_name="c", subcore_axis_name="s")


def _zero_fill(buf):
    zeros16 = jnp.zeros((16,), jnp.float32)

    def zb(i, carry):
        for j in range(D // 16):
            buf[i, pl.ds(j * 16, 16)] = zeros16
        return carry

    lax.fori_loop(0, 128, zb, 0)


@functools.partial(
    pl.kernel,
    out_type=jax.ShapeDtypeStruct((2 * N_PAD, D), jnp.float32),
    mesh=_MESH,
    scratch_types=[
        pltpu.VMEM((NCH, CHUNK), jnp.int32),
        pltpu.VMEM((CHUNK, D), jnp.float32),
        pltpu.VMEM_SHARED((N_PAD, D), jnp.float32),
        pltpu.SemaphoreType.DMA,
    ],
)
def _sc_deg(dstp3, onesr, out, dst_all, ones_v, acc_sh, ssem):
    cid = lax.axis_index("c")
    sid = lax.axis_index("s")
    wid = sid * 2 + cid
    _zero_fill(ones_v)
    for k in range(RPT // CHUNK):
        pltpu.sync_copy(ones_v, acc_sh.at[pl.ds(sid * RPT + k * CHUNK, CHUNK)])
    pltpu.sync_copy(dstp3.at[wid], dst_all)
    pltpu.sync_copy(onesr, ones_v)
    plsc.subcore_barrier()

    def fire(t, carry):
        pltpu.async_copy(ones_v, acc_sh.at[dst_all.at[t]], ssem, add=True)
        return carry

    lax.fori_loop(0, NCH, fire, 0)

    def drain(t, carry):
        pltpu.make_async_copy(ones_v, acc_sh.at[dst_all.at[0]], ssem).wait()
        return carry

    lax.fori_loop(0, NCH, drain, 0)
    plsc.subcore_barrier()
    pltpu.sync_copy(acc_sh.at[pl.ds(sid * RPT, RPT)],
                    out.at[pl.ds(cid * N_PAD + sid * RPT, RPT)])


@functools.partial(
    pl.kernel,
    out_type=jax.ShapeDtypeStruct((2 * N_PAD, D), jnp.float32),
    mesh=_MESH,
    scratch_types=[
        pltpu.VMEM((CP_A, SCHUNK), jnp.int32),
        pltpu.VMEM((CP_A, SCHUNK), jnp.int32),
        pltpu.VMEM((SCHUNK, D), jnp.float32),
        pltpu.VMEM((SCHUNK, D), jnp.float32),
        pltpu.VMEM_SHARED((N_PAD, D), jnp.float32),
        pltpu.SemaphoreType.DMA,
        pltpu.SemaphoreType.DMA,
    ],
)
def _sc_scatter(hs, srcp4, dstp4, out, src_all, dst_all,
                r0, r1, acc_sh, g0, g1):
    rows = (r0, r1)
    gsem = (g0, g1)
    cid = lax.axis_index("c")
    sid = lax.axis_index("s")
    wid = sid * 2 + cid
    _zero_fill(r0)
    for k in range(RPT // SCHUNK):
        pltpu.sync_copy(r0, acc_sh.at[pl.ds(sid * RPT + k * SCHUNK, SCHUNK)])
    plsc.subcore_barrier()

    for p in range(NPASS):
        pltpu.sync_copy(srcp4.at[wid, p], src_all)
        pltpu.sync_copy(dstp4.at[wid, p], dst_all)

        @pl.when(cid == 0)
        def _pipelined():
            pltpu.async_copy(hs.at[src_all.at[0]], rows[0], gsem[0])

            def outer(t2, carry):
                for b in range(2):
                    t = t2 * 2 + b
                    ob = 1 - b
                    pltpu.make_async_copy(hs.at[src_all.at[t]], rows[b],
                                          gsem[b]).wait()

                    @pl.when(t + 1 < CP_A)
                    def _fire():
                        pltpu.async_copy(hs.at[src_all.at[t + 1]], rows[ob],
                                         gsem[ob])

                    pltpu.sync_copy(rows[b], acc_sh.at[dst_all.at[t]],
                                    add=True)
                return carry

            lax.fori_loop(0, CP_A // 2, outer, 0)

        @pl.when(cid == 1)
        def _serial():
            def sbody(t, carry):
                pltpu.async_copy(hs.at[src_all.at[t]], rows[0], gsem[0]).wait()
                pltpu.sync_copy(rows[0], acc_sh.at[dst_all.at[t]], add=True)
                return carry

            lax.fori_loop(0, CP_B, sbody, 0)

    plsc.subcore_barrier()
    pltpu.sync_copy(acc_sh.at[pl.ds(sid * RPT, RPT)],
                    out.at[pl.ds(cid * N_PAD + sid * RPT, RPT)])


@functools.partial(
    pl.kernel,
    out_type=jax.ShapeDtypeStruct((UV, D), jnp.float32),
    mesh=_MESH,
    scratch_types=[
        pltpu.VMEM((UV_NCH, CHUNK), jnp.int32),
        pltpu.VMEM((CHUNK, D), jnp.float32),
        pltpu.VMEM((CHUNK, D), jnp.float32),
        pltpu.VMEM((CHUNK, D), jnp.float32),
        pltpu.VMEM((CHUNK, D), jnp.float32),
        pltpu.SemaphoreType.DMA,
        pltpu.SemaphoreType.DMA,
        pltpu.SemaphoreType.DMA,
        pltpu.SemaphoreType.DMA,
        pltpu.SemaphoreType.DMA,
        pltpu.SemaphoreType.DMA,
        pltpu.SemaphoreType.DMA,
        pltpu.SemaphoreType.DMA,
    ],
)
def _sc_gather(h3, idx3, out, idx_all, r0, r1, r2, r3,
               g0, g1, g2, g3, w0, w1, w2, w3):
    rows = (r0, r1, r2, r3)
    gsem = (g0, g1, g2, g3)
    wsem = (w0, w1, w2, w3)
    cid = lax.axis_index("c")
    sid = lax.axis_index("s")
    wid = sid * 2 + cid
    pltpu.sync_copy(idx3.at[wid], idx_all)
    for b in range(3):
        pltpu.async_copy(h3.at[idx_all.at[b]], rows[b], gsem[b])

    def outer(t2, carry):
        for b in range(NBUF):
            t = t2 * NBUF + b
            b3 = (b + 3) % NBUF
            pltpu.make_async_copy(h3.at[idx_all.at[t]], rows[b], gsem[b]).wait()
            pltpu.async_copy(rows[b],
                             out.at[pl.ds(wid * UV_PT + t * CHUNK, CHUNK)],
                             wsem[b])

            @pl.when(t >= 1)
            def _drain():
                pltpu.make_async_copy(
                    rows[b3], out.at[pl.ds(wid * UV_PT, CHUNK)],
                    wsem[b3]).wait()

            @pl.when(t + 3 < UV_NCH)
            def _fire():
                pltpu.async_copy(h3.at[idx_all.at[t + 3]], rows[b3], gsem[b3])
        return carry

    lax.fori_loop(0, UV_NCH // NBUF, outer, 0)
    pltpu.make_async_copy(rows[(UV_NCH - 1) % NBUF],
                          out.at[pl.ds(wid * UV_PT, CHUNK)],
                          wsem[(UV_NCH - 1) % NBUF]).wait()


GB = 8
RB = N_PAD // GB
RB2 = IDX_PAD // GB


def _dinv_col(degp):
    return lax.rsqrt(degp[0] + degp[1] + 1.0)


def _tenc_body(x_ref, ew, eb, h_ref):
    h_ref[...] = (jnp.dot(x_ref[...], ew[...],
                          preferred_element_type=jnp.float32) + eb[...])


def _tenc(x_pad, enc_W, enc_b2):
    return pl.pallas_call(
        _tenc_body,
        grid=(GB,),
        in_specs=[
            pl.BlockSpec((RB, D), lambda i: (i, 0)),
            pl.BlockSpec((D, D), lambda i: (0, 0)),
            pl.BlockSpec((1, D), lambda i: (0, 0)),
        ],
        out_specs=pl.BlockSpec((RB, D), lambda i: (i, 0)),
        out_shape=jax.ShapeDtypeStruct((N_PAD, D), jnp.float32),
    )(x_pad, enc_W, enc_b2)


def _t0_body(h_ref, w0, degp, hs_ref):
    dinv = _dinv_col(degp)
    hs_ref[...] = dinv * jnp.dot(h_ref[...], w0[...],
                                 preferred_element_type=jnp.float32)


def _t0(h, W0, degp):
    return pl.pallas_call(
        _t0_body,
        grid=(GB,),
        in_specs=[
            pl.BlockSpec((RB, D), lambda i: (i, 0)),
            pl.BlockSpec((D, D), lambda i: (0, 0)),
            pl.BlockSpec((2, RB, 1), lambda i: (0, i, 0)),
        ],
        out_specs=pl.BlockSpec((RB, D), lambda i: (i, 0)),
        out_shape=jax.ShapeDtypeStruct((N_PAD, D), jnp.float32),
    )(h, W0, degp)


def _layer_math(sp_ref, hs_ref, h_ref, degp, cb, g, b, m, v):
    dinv = _dinv_col(degp)
    S = sp_ref[0] + sp_ref[1]
    pre = dinv * (S + hs_ref[...]) + cb[...]
    inv_std = lax.rsqrt(v[...] + 1e-5)
    bn = (pre - m[...]) * inv_std * g[...] + b[...]
    return jnp.maximum(bn, 0.0) + h_ref[...], dinv


def _tl_body(sp_ref, hs_ref, h_ref, degp, cb, g, b, m, v, wn, hn_ref, hsn_ref):
    hn, dinv = _layer_math(sp_ref, hs_ref, h_ref, degp, cb, g, b, m, v)
    hn_ref[...] = hn
    hsn_ref[...] = dinv * jnp.dot(hn, wn[...],
                                  preferred_element_type=jnp.float32)


def _tl_last_body(sp_ref, hs_ref, h_ref, degp, cb, g, b, m, v, hn_ref):
    hn, _ = _layer_math(sp_ref, hs_ref, h_ref, degp, cb, g, b, m, v)
    hn_ref[...] = hn


_VEC_SPEC = pl.BlockSpec((1, D), lambda i: (0, 0))


def _tl(Sp, hs, h, degp, cb, g, b, m, v, Wn):
    return pl.pallas_call(
        _tl_body,
        grid=(GB,),
        in_specs=[
            pl.BlockSpec((2, RB, D), lambda i: (0, i, 0)),
            pl.BlockSpec((RB, D), lambda i: (i, 0)),
            pl.BlockSpec((RB, D), lambda i: (i, 0)),
            pl.BlockSpec((2, RB, 1), lambda i: (0, i, 0)),
            _VEC_SPEC, _VEC_SPEC, _VEC_SPEC, _VEC_SPEC, _VEC_SPEC,
            pl.BlockSpec((D, D), lambda i: (0, 0)),
        ],
        out_specs=[pl.BlockSpec((RB, D), lambda i: (i, 0))] * 2,
        out_shape=[jax.ShapeDtypeStruct((N_PAD, D), jnp.float32)] * 2,
    )(Sp, hs, h, degp, cb, g, b, m, v, Wn)


def _tl_last(Sp, hs, h, degp, cb, g, b, m, v):
    return pl.pallas_call(
        _tl_last_body,
        grid=(GB,),
        in_specs=[
            pl.BlockSpec((2, RB, D), lambda i: (0, i, 0)),
            pl.BlockSpec((RB, D), lambda i: (i, 0)),
            pl.BlockSpec((RB, D), lambda i: (i, 0)),
            pl.BlockSpec((2, RB, 1), lambda i: (0, i, 0)),
            _VEC_SPEC, _VEC_SPEC, _VEC_SPEC, _VEC_SPEC, _VEC_SPEC,
        ],
        out_specs=pl.BlockSpec((RB, D), lambda i: (i, 0)),
        out_shape=jax.ShapeDtypeStruct((N_PAD, D), jnp.float32),
    )(Sp, hs, h, degp, cb, g, b, m, v)


def _mlp_body(nu, nv, w1a, w1b, b1, w2, b2, out_ref):
    hid = (jnp.dot(nu[...], w1a[...], preferred_element_type=jnp.float32)
           + jnp.dot(nv[...], w1b[...], preferred_element_type=jnp.float32)
           + b1[...])
    hid = jnp.maximum(hid, 0.0)
    out_ref[...] = jnp.dot(hid, w2[...], preferred_element_type=jnp.float32) + b2[...]


def _mlp(nu, nv, W1a, W1b, b1, W2, b2):
    return pl.pallas_call(
        _mlp_body,
        grid=(GB,),
        in_specs=[
            pl.BlockSpec((RB2, D), lambda i: (i, 0)),
            pl.BlockSpec((RB2, D), lambda i: (i, 0)),
            pl.BlockSpec((D, D), lambda i: (0, 0)),
            pl.BlockSpec((D, D), lambda i: (0, 0)),
            _VEC_SPEC,
            pl.BlockSpec((D, 1), lambda i: (0, 0)),
            pl.BlockSpec((1, 1), lambda i: (0, 0)),
        ],
        out_specs=pl.BlockSpec((RB2, 1), lambda i: (i, 0)),
        out_shape=jax.ShapeDtypeStruct((IDX_PAD, 1), jnp.float32),
    )(nu, nv, W1a, W1b, b1, W2, b2)


def kernel(x, edge_index, num_graphs, branch_u, branch_v, enc_W, enc_b,
           conv_W, conv_b, bn_gamma, bn_beta, bn_mean, bn_var,
           mlp_W1, mlp_b1, mlp_W2, mlp_b2):
    src = edge_index[0]
    dst = edge_index[1]
    pad_e = E_PAD - E
    srcp = jnp.concatenate([src, jnp.zeros((pad_e,), jnp.int32)])
    dstp = jnp.concatenate([dst, jnp.full((pad_e,), N_NODES, jnp.int32)])
    dstp3 = dstp.reshape(N_TILES, NCH, CHUNK)
    def _split_idx(a):
        e0 = 16 * NPASS * CP_A * SCHUNK
        c0b = a[:e0].reshape(16, NPASS, CP_A, SCHUNK)
        c1b = a[e0:].reshape(16, NPASS, CP_B, SCHUNK)
        c1b = jnp.pad(c1b, ((0, 0), (0, 0), (0, CP_A - CP_B), (0, 0)))
        return jnp.stack([c0b, c1b], axis=1).reshape(
            N_TILES, NPASS, CP_A, SCHUNK)

    srcp4 = _split_idx(srcp)
    dstp4 = _split_idx(dstp)
    x_pad = jnp.pad(x, ((0, N_PAD - N_NODES), (0, 0)))
    onesr = jnp.ones((CHUNK, D), jnp.float32)

    degp = _sc_deg(dstp3, onesr).reshape(2, N_PAD, D)[:, :, :1]

    h = _tenc(x_pad, enc_W, enc_b.reshape(1, D))
    hs = _t0(h, conv_W[0], degp)
    for i in range(3):
        Sp = _sc_scatter(hs, srcp4, dstp4).reshape(2, N_PAD, D)
        args = (Sp, hs, h, degp, conv_b[i].reshape(1, D),
                bn_gamma[i].reshape(1, D), bn_beta[i].reshape(1, D),
                bn_mean[i].reshape(1, D), bn_var[i].reshape(1, D))
        if i < 2:
            h, hs = _tl(*args, conv_W[i + 1])
        else:
            h = _tl_last(*args)

    nb = branch_u.shape[0]
    num_graphs_zero = (jnp.asarray(num_graphs) * 0).astype(branch_u.dtype)
    offsets = (jnp.arange(NUM_GRAPHS, dtype=branch_u.dtype) * NODES_PER_GRAPH
               + num_graphs_zero)
    u_idx = (branch_u[None, :] + offsets[:, None]).reshape(-1)
    v_idx = (branch_v[None, :] + offsets[:, None]).reshape(-1)
    nout = NUM_GRAPHS * nb
    pad_i = IDX_PAD - nout
    uv = jnp.concatenate([
        u_idx, jnp.zeros((pad_i,), branch_u.dtype),
        v_idx, jnp.zeros((pad_i,), branch_u.dtype),
    ]).reshape(N_TILES, UV_NCH, CHUNK)
    gth = _sc_gather(h, uv)
    out_full = _mlp(gth[:IDX_PAD], gth[IDX_PAD:], mlp_W1[:D], mlp_W1[D:],
                    mlp_b1.reshape(1, D), mlp_W2, mlp_b2.reshape(1, 1))
    return out_full[:nout]
